# Initial kernel scaffold; baseline (speedup 1.0000x reference)
import jax
import jax.numpy as jnp
from jax.experimental import pallas as pl


def kernel(x, edge_index, c1_Wl, c1_bl, c1_Wr, c1_br, c1_att, c1_bias, c2_Wl, c2_bl, c2_Wr, c2_br, c2_att, c2_bias, c3_Wl, c3_bl, c3_Wr, c3_br, c3_att, c3_bias, c4_Wl, c4_bl, c4_Wr, c4_br, c4_att, c4_bias, c5_Wl, c5_bl, c5_Wr, c5_br, c5_att, c5_bias, r1_W, r1_b, r2_W, r2_b, r3_W, r3_b, r4_W, r4_b):
    raise NotImplementedError("write your pallas kernel here")


# speedup vs baseline: 39.6654x; 39.6654x over previous
import functools
import numpy as np
import jax
import jax.numpy as jnp
from jax import lax
from jax.experimental import pallas as pl
from jax.experimental.pallas import tpu as pltpu
from jax.experimental.pallas import tpu_sc as plsc

N = 10000
E = 320000
D = 128
H = 8
HID = 8
NC = 40

NCORES, NSUB, L = 2, 16, 16
NW = NCORES * NSUB
NPAD = 10240
BLK = 128
EPAD = 331776
NB32 = EPAD // (NW * BLK)
NB16 = EPAD // (NSUB * BLK)
RB = 1280
GRID = NPAD // RB

_PERM64 = np.array([(f % H) * HID + f // H for f in range(H * HID)], dtype=np.int32)
W5 = 48


def _fold(s, idx):
    return s + s.at[idx].get(mode="promise_in_bounds")


def _sc_edge_kernel(nvec, nfold, srcI, dstI, att_hbm, xl_hbm, xr_hbm, out_hbm,
                    src_v, dst_v, attv, xlg, xrg, wv, srcg, dstg,
                    sem1, sem2, acc_sh):
    cid = lax.axis_index("c")
    sid = lax.axis_index("s")
    per_core = srcI.shape[0] == NSUB
    nb = srcI.shape[1]
    if per_core:
        pltpu.sync_copy(srcI.at[sid], src_v)
        pltpu.sync_copy(dstI.at[sid], dst_v)
        pltpu.sync_copy(att_hbm.at[pl.ds(cid * nvec * L, nvec * L)], attv)
    else:
        wid = sid * NCORES + cid
        pltpu.sync_copy(srcI.at[wid], src_v)
        pltpu.sync_copy(dstI.at[wid], dst_v)
        pltpu.sync_copy(att_hbm, attv)

    zero = jnp.zeros((L,), jnp.float32)

    def zrow(e, _):
        for k in range(nvec + 1):
            wv[e, pl.ds(L * k, L)] = zero
        return _

    lax.fori_loop(0, BLK, zrow, None)
    rows = NPAD // NSUB
    for k in range(rows // BLK):
        pltpu.sync_copy(wv, acc_sh.at[pl.ds(sid * rows + k * BLK, BLK)])
    plsc.subcore_barrier()

    attk = [attv[pl.ds(L * k, L)] for k in range(nvec)]
    iota = jnp.arange(L, dtype=jnp.int32)
    rots = [(iota + r) % L for r in (8, 4, 2, 1)][:nfold]

    def block(b, _):
        if per_core:
            off = cid * NPAD
            for k in range(BLK // L):
                srcg[pl.ds(L * k, L)] = src_v[b, pl.ds(L * k, L)] + off
                dstg[pl.ds(L * k, L)] = dst_v[b, pl.ds(L * k, L)] + off
            cp1 = pltpu.async_copy(xl_hbm.at[srcg], xlg, sem1)
            cp2 = pltpu.async_copy(xr_hbm.at[dstg], xrg, sem2)
        else:
            cp1 = pltpu.async_copy(xl_hbm.at[src_v.at[b]], xlg, sem1)
            cp2 = pltpu.async_copy(xr_hbm.at[dst_v.at[b]], xrg, sem2)
        cp1.wait()
        cp2.wait()

        def edge(e, _):
            xs = []
            s = None
            for k in range(nvec):
                a = xlg[e, pl.ds(L * k, L)]
                t = a + xrg[e, pl.ds(L * k, L)]
                u = jnp.maximum(t, 0.2 * t)
                xs.append(a)
                p = u * attk[k]
                s = p if s is None else s + p
            for r in rots:
                s = _fold(s, r)
            ex = jnp.exp(s)
            for k in range(nvec):
                wv[e, pl.ds(L * k, L)] = xs[k] * ex
            wv[e, pl.ds(L * nvec, L)] = ex
            return _

        lax.fori_loop(0, BLK, edge, None)
        pltpu.sync_copy(wv, acc_sh.at[dst_v.at[b]], add=True)
        return _

    lax.fori_loop(0, nb, block, None)
    plsc.subcore_barrier()
    pltpu.sync_copy(acc_sh.at[pl.ds(sid * rows, rows)],
                    out_hbm.at[pl.ds(cid * NPAD + sid * rows, rows)])


def _sc_edge(xl, xr, attp, srcI, dstI, nvec, nfold, per_core):
    width = nvec * L + L
    mesh = plsc.VectorSubcoreMesh(core_axis_name="c", subcore_axis_name="s")
    nb = NB16 if per_core else NB32
    fn = pl.kernel(
        functools.partial(_sc_edge_kernel, nvec, nfold),
        out_type=jax.ShapeDtypeStruct((NCORES * NPAD, width), jnp.float32),
        mesh=mesh,
        compiler_params=pltpu.CompilerParams(use_tc_tiling_on_sc=False),
        scratch_types=[
            pltpu.VMEM((nb, BLK), jnp.int32),
            pltpu.VMEM((nb, BLK), jnp.int32),
            pltpu.VMEM((nvec * L,), jnp.float32),
            pltpu.VMEM((BLK, nvec * L), jnp.float32),
            pltpu.VMEM((BLK, nvec * L), jnp.float32),
            pltpu.VMEM((BLK, width), jnp.float32),
            pltpu.VMEM((BLK,), jnp.int32),
            pltpu.VMEM((BLK,), jnp.int32),
            pltpu.SemaphoreType.DMA,
            pltpu.SemaphoreType.DMA,
            pltpu.VMEM_SHARED((NPAD, width), jnp.float32),
        ],
    )
    return fn(srcI, dstI, attp, xl, xr).reshape(NCORES, NPAD, width)


def _tc_proj_kernel(x, Wl, bl, Wr, br, xl, xr):
    h = x[...]
    xl[...] = jnp.dot(h, Wl[...], preferred_element_type=jnp.float32) + bl[...]
    xr[...] = jnp.dot(h, Wr[...], preferred_element_type=jnp.float32) + br[...]


def _tc_proj(x, Wl, bl, Wr, br):
    din, w = Wl.shape
    return pl.pallas_call(
        _tc_proj_kernel,
        grid=(GRID,),
        in_specs=[
            pl.BlockSpec((RB, din), lambda i: (i, 0)),
            pl.BlockSpec((din, w), lambda i: (0, 0)),
            pl.BlockSpec((1, w), lambda i: (0, 0)),
            pl.BlockSpec((din, w), lambda i: (0, 0)),
            pl.BlockSpec((1, w), lambda i: (0, 0)),
        ],
        out_specs=[
            pl.BlockSpec((RB, w), lambda i: (i, 0)),
            pl.BlockSpec((RB, w), lambda i: (i, 0)),
        ],
        out_shape=[jax.ShapeDtypeStruct((NPAD, w), jnp.float32)] * 2,
    )(x, Wl, bl.reshape(1, -1), Wr, br.reshape(1, -1))


def _tc_mid_kernel(acc, h, rW, rb, cb, Wl, bl, Wr, br, hnew, xl, xr):
    a = acc[0] + acc[1]
    num = a[:, : H * HID]
    den = a[:, H * HID : H * HID + H]
    denb = jnp.reshape(
        jnp.broadcast_to(den.reshape(RB, 1, H), (RB, HID, H)), (RB, H * HID))
    o = num / (denb + 1e-16) + cb[...]
    hp = h[...]
    res = jnp.dot(hp, rW[...], preferred_element_type=jnp.float32) + rb[...]
    t = o + res
    hn = jnp.maximum(t, 0.2 * t)
    hnew[...] = hn
    xl[...] = jnp.dot(hn, Wl[...], preferred_element_type=jnp.float32) + bl[...]
    xr[...] = jnp.dot(hn, Wr[...], preferred_element_type=jnp.float32) + br[...]


def _tc_mid(acc, h, rW, rb, cb, Wl, bl, Wr, br):
    din = h.shape[1]
    w = Wl.shape[1]
    hd = H * HID
    return pl.pallas_call(
        _tc_mid_kernel,
        grid=(GRID,),
        in_specs=[
            pl.BlockSpec((2, RB, hd + L), lambda i: (0, i, 0)),
            pl.BlockSpec((RB, din), lambda i: (i, 0)),
            pl.BlockSpec((din, hd), lambda i: (0, 0)),
            pl.BlockSpec((1, hd), lambda i: (0, 0)),
            pl.BlockSpec((1, hd), lambda i: (0, 0)),
            pl.BlockSpec((hd, w), lambda i: (0, 0)),
            pl.BlockSpec((1, w), lambda i: (0, 0)),
            pl.BlockSpec((hd, w), lambda i: (0, 0)),
            pl.BlockSpec((1, w), lambda i: (0, 0)),
        ],
        out_specs=[
            pl.BlockSpec((RB, hd), lambda i: (i, 0)),
            pl.BlockSpec((RB, w), lambda i: (i, 0)),
            pl.BlockSpec((RB, w), lambda i: (i, 0)),
        ],
        out_shape=[
            jax.ShapeDtypeStruct((NPAD, hd), jnp.float32),
            jax.ShapeDtypeStruct((NPAD, w), jnp.float32),
            jax.ShapeDtypeStruct((NPAD, w), jnp.float32),
        ],
    )(acc, h, rW, rb.reshape(1, -1), cb.reshape(1, -1),
      Wl, bl.reshape(1, -1), Wr, br.reshape(1, -1))


def _tc_final_kernel(a0, a1, a2, a3, cb, out):
    s = None
    for acc in (a0, a1, a2, a3):
        for c in (0, 1):
            a = acc[c]
            num = a[:, :NC]
            den = a[:, W5 : W5 + 1]
            q = num / (den + 1e-16)
            s = q if s is None else s + q
    logits = s * 0.125 + cb[...]
    m = jnp.max(logits, axis=1, keepdims=True)
    z = logits - m
    out[...] = z - jnp.log(jnp.sum(jnp.exp(z), axis=1, keepdims=True))


def _tc_final(accs, cb):
    w5 = W5 + L
    return pl.pallas_call(
        _tc_final_kernel,
        grid=(GRID,),
        in_specs=[pl.BlockSpec((2, RB, w5), lambda i: (0, i, 0))] * 4 + [
            pl.BlockSpec((1, NC), lambda i: (0, 0)),
        ],
        out_specs=pl.BlockSpec((RB, NC), lambda i: (i, 0)),
        out_shape=jax.ShapeDtypeStruct((NPAD, NC), jnp.float32),
    )(*accs, cb.reshape(1, -1))


@jax.jit
def kernel(x, edge_index,
           c1_Wl, c1_bl, c1_Wr, c1_br, c1_att, c1_bias,
           c2_Wl, c2_bl, c2_Wr, c2_br, c2_att, c2_bias,
           c3_Wl, c3_bl, c3_Wr, c3_br, c3_att, c3_bias,
           c4_Wl, c4_bl, c4_Wr, c4_br, c4_att, c4_bias,
           c5_Wl, c5_bl, c5_Wr, c5_br, c5_att, c5_bias,
           r1_W, r1_b, r2_W, r2_b, r3_W, r3_b, r4_W, r4_b):
    p = dict(locals())
    perm = _PERM64
    loop = jnp.arange(N, dtype=jnp.int32)
    pad = jnp.full((EPAD - E - N,), N, jnp.int32)
    src = jnp.concatenate([edge_index[0], loop, pad])
    dst = jnp.concatenate([edge_index[1], loop, pad])
    srcI32 = src.reshape(NW, NB32, BLK)
    dstI32 = dst.reshape(NW, NB32, BLK)
    srcI16 = src.reshape(NSUB, NB16, BLK)
    dstI16 = dst.reshape(NSUB, NB16, BLK)
    xpad = jnp.pad(x, ((0, NPAD - N), (0, 0)))

    att_p = []
    for i in range(1, 5):
        att = p[f"c{i}_att"]
        att_p.append(att.T.reshape(-1))
    att5 = p["c5_att"]
    zpad8 = jnp.zeros((8,), jnp.float32)
    att5_h = [jnp.concatenate([att5[hh], zpad8]) for hh in range(H)]

    h = xpad
    for i in range(1, 5):
        Wl = p[f"c{i}_Wl"][:, perm]
        bl = p[f"c{i}_bl"][perm]
        Wr = p[f"c{i}_Wr"][:, perm]
        br = p[f"c{i}_br"][perm]
        if i > 1:
            Wl = Wl[perm]
            Wr = Wr[perm]
        if i == 1:
            xl, xr = _tc_proj(h, Wl, bl, Wr, br)
        acc = _sc_edge(xl, xr, att_p[i - 1], srcI32, dstI32, 4, 1, False)
        cb = p[f"c{i}_bias"][perm]
        rW = p[f"r{i}_W"][:, perm]
        if i > 1:
            rW = rW[perm]
        rb = p[f"r{i}_b"][perm]
        if i < 4:
            nWl = p[f"c{i+1}_Wl"][:, perm][perm]
            nbl = p[f"c{i+1}_bl"][perm]
            nWr = p[f"c{i+1}_Wr"][:, perm][perm]
            nbr = p[f"c{i+1}_br"][perm]
        else:
            Wl5 = p["c5_Wl"][perm]
            Wr5 = p["c5_Wr"][perm]
            zW = jnp.zeros((H * HID, 8), jnp.float32)
            nWl = jnp.concatenate(
                sum(([Wl5[:, hh * NC : (hh + 1) * NC], zW] for hh in range(H)),
                    []), axis=1)
            nWr = jnp.concatenate(
                sum(([Wr5[:, hh * NC : (hh + 1) * NC], zW] for hh in range(H)),
                    []), axis=1)
            nbl = jnp.concatenate(
                sum(([p["c5_bl"][hh * NC : (hh + 1) * NC], zpad8]
                     for hh in range(H)), []))
            nbr = jnp.concatenate(
                sum(([p["c5_br"][hh * NC : (hh + 1) * NC], zpad8]
                     for hh in range(H)), []))
        h, xl, xr = _tc_mid(acc, h, rW, rb, cb, nWl, nbl, nWr, nbr)

    accs = []
    for j in range(4):
        xlj = jnp.concatenate([xl[:, (2 * j) * W5 : (2 * j + 1) * W5],
                               xl[:, (2 * j + 1) * W5 : (2 * j + 2) * W5]],
                              axis=0)
        xrj = jnp.concatenate([xr[:, (2 * j) * W5 : (2 * j + 1) * W5],
                               xr[:, (2 * j + 1) * W5 : (2 * j + 2) * W5]],
                              axis=0)
        attj = jnp.concatenate([att5_h[2 * j], att5_h[2 * j + 1]])
        accs.append(_sc_edge(xlj, xrj, attj, srcI16, dstI16, 3, 4, True))
    out = _tc_final(accs, p["c5_bias"])
    return out[:N]


# speedup vs baseline: 50.7823x; 1.2803x over previous
import functools
import numpy as np
import jax
import jax.numpy as jnp
from jax import lax
from jax.experimental import pallas as pl
from jax.experimental.pallas import tpu as pltpu
from jax.experimental.pallas import tpu_sc as plsc

N = 10000
E = 320000
D = 128
H = 8
HID = 8
NC = 40

NCORES, NSUB, L = 2, 16, 16
NW = NCORES * NSUB
NPAD = 10240
BLK = 128
EPAD = 331776
NB32 = EPAD // (NW * BLK)
NB16 = EPAD // (NSUB * BLK)
RB = 1280
GRID = NPAD // RB

_PERM64 = np.array([(f % H) * HID + f // H for f in range(H * HID)], dtype=np.int32)
W5 = 48


def _fold(s, idx):
    return s + s.at[idx].get(mode="promise_in_bounds")


def _sc_edge_kernel(nvec, nfold, srcI, dstI, att_hbm, xl_hbm, xr_hbm, out_hbm,
                    src_v, dst_v, attv, xlg0, xrg0, xlg1, xrg1, wv,
                    srcg0, dstg0, srcg1, dstg1,
                    s1a, s2a, s1b, s2b, acc_sh):
    cid = lax.axis_index("c")
    sid = lax.axis_index("s")
    per_core = srcI.shape[0] == NSUB
    nb = srcI.shape[1]
    if per_core:
        pltpu.sync_copy(srcI.at[sid], src_v)
        pltpu.sync_copy(dstI.at[sid], dst_v)
        pltpu.sync_copy(att_hbm.at[pl.ds(cid * nvec * L, nvec * L)], attv)
    else:
        wid = sid * NCORES + cid
        pltpu.sync_copy(srcI.at[wid], src_v)
        pltpu.sync_copy(dstI.at[wid], dst_v)
        pltpu.sync_copy(att_hbm, attv)

    zero = jnp.zeros((L,), jnp.float32)

    def zrow(e, _):
        for k in range(nvec + 1):
            wv[e, pl.ds(L * k, L)] = zero
        return _

    lax.fori_loop(0, BLK, zrow, None)
    rows = NPAD // NSUB
    for k in range(rows // BLK):
        pltpu.sync_copy(wv, acc_sh.at[pl.ds(sid * rows + k * BLK, BLK)])
    plsc.subcore_barrier()

    attk = [attv[pl.ds(L * k, L)] for k in range(nvec)]
    iota = jnp.arange(L, dtype=jnp.int32)
    rots = [(iota + r) % L for r in (8, 4, 2, 1)][:nfold]

    xlgs, xrgs = (xlg0, xlg1), (xrg0, xrg1)
    srcgs, dstgs = (srcg0, srcg1), (dstg0, dstg1)
    s1s, s2s = (s1a, s1b), (s2a, s2b)

    def issue(b, j):
        if per_core:
            off = cid * NPAD
            for k in range(BLK // L):
                srcgs[j][pl.ds(L * k, L)] = src_v[b, pl.ds(L * k, L)] + off
                dstgs[j][pl.ds(L * k, L)] = dst_v[b, pl.ds(L * k, L)] + off
            pltpu.async_copy(xl_hbm.at[srcgs[j]], xlgs[j], s1s[j])
            pltpu.async_copy(xr_hbm.at[dstgs[j]], xrgs[j], s2s[j])
        else:
            pltpu.async_copy(xl_hbm.at[src_v.at[b]], xlgs[j], s1s[j])
            pltpu.async_copy(xr_hbm.at[dst_v.at[b]], xrgs[j], s2s[j])

    def wait(j):
        if per_core:
            pltpu.make_async_copy(xl_hbm.at[srcgs[j]], xlgs[j], s1s[j]).wait()
            pltpu.make_async_copy(xr_hbm.at[dstgs[j]], xrgs[j], s2s[j]).wait()
        else:
            pltpu.make_async_copy(
                xl_hbm.at[src_v.at[0]], xlgs[j], s1s[j]).wait()
            pltpu.make_async_copy(
                xr_hbm.at[dst_v.at[0]], xrgs[j], s2s[j]).wait()

    def compute(b, j):
        xlg, xrg = xlgs[j], xrgs[j]

        def edge(e, _):
            xs = []
            s = None
            for k in range(nvec):
                a = xlg[e, pl.ds(L * k, L)]
                t = a + xrg[e, pl.ds(L * k, L)]
                u = jnp.maximum(t, 0.2 * t)
                xs.append(a)
                p = u * attk[k]
                s = p if s is None else s + p
            for r in rots:
                s = _fold(s, r)
            ex = jnp.exp(s)
            for k in range(nvec):
                wv[e, pl.ds(L * k, L)] = xs[k] * ex
            wv[e, pl.ds(L * nvec, L)] = ex
            return _

        lax.fori_loop(0, BLK, edge, None)
        pltpu.sync_copy(wv, acc_sh.at[dst_v.at[b]], add=True)

    last = nb - 1
    issue(0, 0)
    issue(1, 1)

    def body(i, _):
        b0 = 2 * i
        wait(0)
        compute(b0, 0)
        issue(jnp.minimum(b0 + 2, last), 0)
        wait(1)
        compute(b0 + 1, 1)
        issue(jnp.minimum(b0 + 3, last), 1)
        return _

    lax.fori_loop(0, nb // 2, body, None)
    if nb % 2:
        wait(0)
        compute(last, 0)
        wait(1)
    else:
        wait(0)
        wait(1)
    plsc.subcore_barrier()
    pltpu.sync_copy(acc_sh.at[pl.ds(sid * rows, rows)],
                    out_hbm.at[pl.ds(cid * NPAD + sid * rows, rows)])


def _sc_edge(xl, xr, attp, srcI, dstI, nvec, nfold, per_core):
    width = nvec * L + L
    mesh = plsc.VectorSubcoreMesh(core_axis_name="c", subcore_axis_name="s")
    nb = NB16 if per_core else NB32
    fn = pl.kernel(
        functools.partial(_sc_edge_kernel, nvec, nfold),
        out_type=jax.ShapeDtypeStruct((NCORES * NPAD, width), jnp.float32),
        mesh=mesh,
        compiler_params=pltpu.CompilerParams(use_tc_tiling_on_sc=False),
        scratch_types=[
            pltpu.VMEM((nb, BLK), jnp.int32),
            pltpu.VMEM((nb, BLK), jnp.int32),
            pltpu.VMEM((nvec * L,), jnp.float32),
            pltpu.VMEM((BLK, nvec * L), jnp.float32),
            pltpu.VMEM((BLK, nvec * L), jnp.float32),
            pltpu.VMEM((BLK, nvec * L), jnp.float32),
            pltpu.VMEM((BLK, nvec * L), jnp.float32),
            pltpu.VMEM((BLK, width), jnp.float32),
            pltpu.VMEM((BLK,), jnp.int32),
            pltpu.VMEM((BLK,), jnp.int32),
            pltpu.VMEM((BLK,), jnp.int32),
            pltpu.VMEM((BLK,), jnp.int32),
            pltpu.SemaphoreType.DMA,
            pltpu.SemaphoreType.DMA,
            pltpu.SemaphoreType.DMA,
            pltpu.SemaphoreType.DMA,
            pltpu.VMEM_SHARED((NPAD, width), jnp.float32),
        ],
    )
    return fn(srcI, dstI, attp, xl, xr).reshape(NCORES, NPAD, width)


def _tc_proj_kernel(x, Wl, bl, Wr, br, xl, xr):
    h = x[...]
    xl[...] = jnp.dot(h, Wl[...], preferred_element_type=jnp.float32) + bl[...]
    xr[...] = jnp.dot(h, Wr[...], preferred_element_type=jnp.float32) + br[...]


def _tc_proj(x, Wl, bl, Wr, br):
    din, w = Wl.shape
    return pl.pallas_call(
        _tc_proj_kernel,
        grid=(GRID,),
        in_specs=[
            pl.BlockSpec((RB, din), lambda i: (i, 0)),
            pl.BlockSpec((din, w), lambda i: (0, 0)),
            pl.BlockSpec((1, w), lambda i: (0, 0)),
            pl.BlockSpec((din, w), lambda i: (0, 0)),
            pl.BlockSpec((1, w), lambda i: (0, 0)),
        ],
        out_specs=[
            pl.BlockSpec((RB, w), lambda i: (i, 0)),
            pl.BlockSpec((RB, w), lambda i: (i, 0)),
        ],
        out_shape=[jax.ShapeDtypeStruct((NPAD, w), jnp.float32)] * 2,
    )(x, Wl, bl.reshape(1, -1), Wr, br.reshape(1, -1))


def _tc_mid_kernel(acc, h, rW, rb, cb, Wl, bl, Wr, br, hnew, xl, xr):
    a = acc[0] + acc[1]
    num = a[:, : H * HID]
    den = a[:, H * HID : H * HID + H]
    denb = jnp.reshape(
        jnp.broadcast_to(den.reshape(RB, 1, H), (RB, HID, H)), (RB, H * HID))
    o = num / (denb + 1e-16) + cb[...]
    hp = h[...]
    res = jnp.dot(hp, rW[...], preferred_element_type=jnp.float32) + rb[...]
    t = o + res
    hn = jnp.maximum(t, 0.2 * t)
    hnew[...] = hn
    xl[...] = jnp.dot(hn, Wl[...], preferred_element_type=jnp.float32) + bl[...]
    xr[...] = jnp.dot(hn, Wr[...], preferred_element_type=jnp.float32) + br[...]


def _tc_mid(acc, h, rW, rb, cb, Wl, bl, Wr, br):
    din = h.shape[1]
    w = Wl.shape[1]
    hd = H * HID
    return pl.pallas_call(
        _tc_mid_kernel,
        grid=(GRID,),
        in_specs=[
            pl.BlockSpec((2, RB, hd + L), lambda i: (0, i, 0)),
            pl.BlockSpec((RB, din), lambda i: (i, 0)),
            pl.BlockSpec((din, hd), lambda i: (0, 0)),
            pl.BlockSpec((1, hd), lambda i: (0, 0)),
            pl.BlockSpec((1, hd), lambda i: (0, 0)),
            pl.BlockSpec((hd, w), lambda i: (0, 0)),
            pl.BlockSpec((1, w), lambda i: (0, 0)),
            pl.BlockSpec((hd, w), lambda i: (0, 0)),
            pl.BlockSpec((1, w), lambda i: (0, 0)),
        ],
        out_specs=[
            pl.BlockSpec((RB, hd), lambda i: (i, 0)),
            pl.BlockSpec((RB, w), lambda i: (i, 0)),
            pl.BlockSpec((RB, w), lambda i: (i, 0)),
        ],
        out_shape=[
            jax.ShapeDtypeStruct((NPAD, hd), jnp.float32),
            jax.ShapeDtypeStruct((NPAD, w), jnp.float32),
            jax.ShapeDtypeStruct((NPAD, w), jnp.float32),
        ],
    )(acc, h, rW, rb.reshape(1, -1), cb.reshape(1, -1),
      Wl, bl.reshape(1, -1), Wr, br.reshape(1, -1))


def _tc_final_kernel(a0, a1, a2, a3, cb, out):
    s = None
    for acc in (a0, a1, a2, a3):
        for c in (0, 1):
            a = acc[c]
            num = a[:, :NC]
            den = a[:, W5 : W5 + 1]
            q = num / (den + 1e-16)
            s = q if s is None else s + q
    logits = s * 0.125 + cb[...]
    m = jnp.max(logits, axis=1, keepdims=True)
    z = logits - m
    out[...] = z - jnp.log(jnp.sum(jnp.exp(z), axis=1, keepdims=True))


def _tc_final(accs, cb):
    w5 = W5 + L
    return pl.pallas_call(
        _tc_final_kernel,
        grid=(GRID,),
        in_specs=[pl.BlockSpec((2, RB, w5), lambda i: (0, i, 0))] * 4 + [
            pl.BlockSpec((1, NC), lambda i: (0, 0)),
        ],
        out_specs=pl.BlockSpec((RB, NC), lambda i: (i, 0)),
        out_shape=jax.ShapeDtypeStruct((NPAD, NC), jnp.float32),
    )(*accs, cb.reshape(1, -1))


@jax.jit
def kernel(x, edge_index,
           c1_Wl, c1_bl, c1_Wr, c1_br, c1_att, c1_bias,
           c2_Wl, c2_bl, c2_Wr, c2_br, c2_att, c2_bias,
           c3_Wl, c3_bl, c3_Wr, c3_br, c3_att, c3_bias,
           c4_Wl, c4_bl, c4_Wr, c4_br, c4_att, c4_bias,
           c5_Wl, c5_bl, c5_Wr, c5_br, c5_att, c5_bias,
           r1_W, r1_b, r2_W, r2_b, r3_W, r3_b, r4_W, r4_b):
    p = dict(locals())
    perm = _PERM64
    loop = jnp.arange(N, dtype=jnp.int32)
    pad = jnp.full((EPAD - E - N,), N, jnp.int32)
    src = jnp.concatenate([edge_index[0], loop, pad])
    dst = jnp.concatenate([edge_index[1], loop, pad])
    srcI32 = src.reshape(NW, NB32, BLK)
    dstI32 = dst.reshape(NW, NB32, BLK)
    srcI16 = src.reshape(NSUB, NB16, BLK)
    dstI16 = dst.reshape(NSUB, NB16, BLK)
    xpad = jnp.pad(x, ((0, NPAD - N), (0, 0)))

    att_p = []
    for i in range(1, 5):
        att = p[f"c{i}_att"]
        att_p.append(att.T.reshape(-1))
    att5 = p["c5_att"]
    zpad8 = jnp.zeros((8,), jnp.float32)
    att5_h = [jnp.concatenate([att5[hh], zpad8]) for hh in range(H)]

    h = xpad
    for i in range(1, 5):
        Wl = p[f"c{i}_Wl"][:, perm]
        bl = p[f"c{i}_bl"][perm]
        Wr = p[f"c{i}_Wr"][:, perm]
        br = p[f"c{i}_br"][perm]
        if i > 1:
            Wl = Wl[perm]
            Wr = Wr[perm]
        if i == 1:
            xl, xr = _tc_proj(h, Wl, bl, Wr, br)
        acc = _sc_edge(xl, xr, att_p[i - 1], srcI32, dstI32, 4, 1, False)
        cb = p[f"c{i}_bias"][perm]
        rW = p[f"r{i}_W"][:, perm]
        if i > 1:
            rW = rW[perm]
        rb = p[f"r{i}_b"][perm]
        if i < 4:
            nWl = p[f"c{i+1}_Wl"][:, perm][perm]
            nbl = p[f"c{i+1}_bl"][perm]
            nWr = p[f"c{i+1}_Wr"][:, perm][perm]
            nbr = p[f"c{i+1}_br"][perm]
        else:
            Wl5 = p["c5_Wl"][perm]
            Wr5 = p["c5_Wr"][perm]
            zW = jnp.zeros((H * HID, 8), jnp.float32)
            nWl = jnp.concatenate(
                sum(([Wl5[:, hh * NC : (hh + 1) * NC], zW] for hh in range(H)),
                    []), axis=1)
            nWr = jnp.concatenate(
                sum(([Wr5[:, hh * NC : (hh + 1) * NC], zW] for hh in range(H)),
                    []), axis=1)
            nbl = jnp.concatenate(
                sum(([p["c5_bl"][hh * NC : (hh + 1) * NC], zpad8]
                     for hh in range(H)), []))
            nbr = jnp.concatenate(
                sum(([p["c5_br"][hh * NC : (hh + 1) * NC], zpad8]
                     for hh in range(H)), []))
        h, xl, xr = _tc_mid(acc, h, rW, rb, cb, nWl, nbl, nWr, nbr)

    accs = []
    for j in range(4):
        xlj = jnp.concatenate([xl[:, (2 * j) * W5 : (2 * j + 1) * W5],
                               xl[:, (2 * j + 1) * W5 : (2 * j + 2) * W5]],
                              axis=0)
        xrj = jnp.concatenate([xr[:, (2 * j) * W5 : (2 * j + 1) * W5],
                               xr[:, (2 * j + 1) * W5 : (2 * j + 2) * W5]],
                              axis=0)
        attj = jnp.concatenate([att5_h[2 * j], att5_h[2 * j + 1]])
        accs.append(_sc_edge(xlj, xrj, attj, srcI16, dstI16, 3, 4, True))
    out = _tc_final(accs, p["c5_bias"])
    return out[:N]


# speedup vs baseline: 50.9859x; 1.0040x over previous
import functools
import numpy as np
import jax
import jax.numpy as jnp
from jax import lax
from jax.experimental import pallas as pl
from jax.experimental.pallas import tpu as pltpu
from jax.experimental.pallas import tpu_sc as plsc

N = 10000
E = 320000
D = 128
H = 8
HID = 8
NC = 40

NCORES, NSUB, L = 2, 16, 16
NW = NCORES * NSUB
NPAD = 10240
BLK = 128
EPAD = 331776
NB32 = EPAD // (NW * BLK)
NB16 = EPAD // (NSUB * BLK)
RB = 1280
GRID = NPAD // RB

_PERM64 = np.array([(f % H) * HID + f // H for f in range(H * HID)], dtype=np.int32)
W5 = 48


def _fold(s, idx):
    return s + s.at[idx].get(mode="promise_in_bounds")


def _sc_edge_kernel(nvec, nfold, per_core, srcI, dstI, att_hbm,
                    xl_hbm, xr_hbm, out_hbm, src_v, dst_v, dstg0, dstg1,
                    attv, xlg0, xrg0, xlg1, xrg1, wv,
                    s1a, s2a, s1b, s2b, acc_sh):
    cid = lax.axis_index("c")
    sid = lax.axis_index("s")
    nb = srcI.shape[-2]
    off = cid * NPAD
    if per_core:
        pltpu.sync_copy(srcI.at[sid], src_v)
        pltpu.sync_copy(dstI.at[sid], dst_v)
        pltpu.sync_copy(att_hbm.at[pl.ds(cid * nvec * L, nvec * L)], attv)
        def opre(b, _):
            for k in range(BLK // L):
                sl = pl.ds(L * k, L)
                src_v[b, sl] = src_v[b, sl] + off
            return _

        lax.fori_loop(0, nb, opre, None)
    else:
        wid = sid * NCORES + cid
        pltpu.sync_copy(srcI.at[wid], src_v)
        pltpu.sync_copy(dstI.at[wid], dst_v)
        pltpu.sync_copy(att_hbm, attv)

    nw = nvec + 1
    zero = jnp.zeros((L,), jnp.float32)

    def zrow(e, _):
        for k in range(nw):
            wv[e, pl.ds(L * k, L)] = zero
        return _

    lax.fori_loop(0, BLK, zrow, None)
    rows = NPAD // NSUB
    for k in range(rows // BLK):
        pltpu.sync_copy(wv, acc_sh.at[pl.ds(sid * rows + k * BLK, BLK)])
    plsc.subcore_barrier()

    attk = [attv[pl.ds(L * k, L)] for k in range(nvec)]
    iota = jnp.arange(L, dtype=jnp.int32)
    rots = [(iota + r) % L for r in (8, 4, 2, 1)][:nfold]

    xlgs, xrgs = (xlg0, xlg1), (xrg0, xrg1)
    dstgs = (dstg0, dstg1)
    s1s, s2s = (s1a, s1b), (s2a, s2b)

    def issue(b, j):
        pltpu.async_copy(xl_hbm.at[src_v.at[b]], xlgs[j], s1s[j])
        if per_core:
            for k in range(BLK // L):
                sl = pl.ds(L * k, L)
                dstgs[j][sl] = dst_v[b, sl] + off
            pltpu.async_copy(xr_hbm.at[dstgs[j]], xrgs[j], s2s[j])
        else:
            pltpu.async_copy(xr_hbm.at[dst_v.at[b]], xrgs[j], s2s[j])

    def wait(j):
        pltpu.make_async_copy(xl_hbm.at[src_v.at[0]], xlgs[j], s1s[j]).wait()
        if per_core:
            pltpu.make_async_copy(
                xr_hbm.at[dstgs[j]], xrgs[j], s2s[j]).wait()
        else:
            pltpu.make_async_copy(
                xr_hbm.at[dst_v.at[0]], xrgs[j], s2s[j]).wait()

    def compute(b, j):
        xlg, xrg = xlgs[j], xrgs[j]

        def edge(e, _):
            xs = []
            s = None
            for k in range(nvec):
                a = xlg[e, pl.ds(L * k, L)]
                t = a + xrg[e, pl.ds(L * k, L)]
                u = jnp.maximum(t, 0.2 * t)
                xs.append(a)
                p = u * attk[k]
                s = p if s is None else s + p
            for r in rots:
                s = _fold(s, r)
            ex = jnp.exp(s)
            for k in range(nvec):
                wv[e, pl.ds(L * k, L)] = xs[k] * ex
            wv[e, pl.ds(L * nvec, L)] = ex
            return _

        lax.fori_loop(0, BLK, edge, None)
        pltpu.sync_copy(wv, acc_sh.at[dst_v.at[b]], add=True)

    last = nb - 1
    issue(0, 0)
    issue(1, 1)

    def body(i, _):
        b0 = 2 * i
        wait(0)
        compute(b0, 0)
        issue(jnp.minimum(b0 + 2, last), 0)
        wait(1)
        compute(b0 + 1, 1)
        issue(jnp.minimum(b0 + 3, last), 1)
        return _

    lax.fori_loop(0, nb // 2, body, None)
    if nb % 2:
        wait(0)
        compute(last, 0)
        wait(1)
    else:
        wait(0)
        wait(1)
    plsc.subcore_barrier()
    pltpu.sync_copy(acc_sh.at[pl.ds(sid * rows, rows)],
                    out_hbm.at[pl.ds(cid * NPAD + sid * rows, rows)])


def _sc_edge(xl, xr, attp, srcI, dstI, nvec, nfold, per_core):
    width = nvec * L + L
    mesh = plsc.VectorSubcoreMesh(core_axis_name="c", subcore_axis_name="s")
    nb = NB16 if per_core else NB32
    fn = pl.kernel(
        functools.partial(_sc_edge_kernel, nvec, nfold, per_core),
        out_type=jax.ShapeDtypeStruct((NCORES * NPAD, width), jnp.float32),
        mesh=mesh,
        compiler_params=pltpu.CompilerParams(use_tc_tiling_on_sc=False),
        scratch_types=[
            pltpu.VMEM((nb, BLK), jnp.int32),
            pltpu.VMEM((nb, BLK), jnp.int32),
            pltpu.VMEM((BLK,), jnp.int32),
            pltpu.VMEM((BLK,), jnp.int32),
            pltpu.VMEM((nvec * L,), jnp.float32),
            pltpu.VMEM((BLK, nvec * L), jnp.float32),
            pltpu.VMEM((BLK, nvec * L), jnp.float32),
            pltpu.VMEM((BLK, nvec * L), jnp.float32),
            pltpu.VMEM((BLK, nvec * L), jnp.float32),
            pltpu.VMEM((BLK, width), jnp.float32),
            pltpu.SemaphoreType.DMA,
            pltpu.SemaphoreType.DMA,
            pltpu.SemaphoreType.DMA,
            pltpu.SemaphoreType.DMA,
            pltpu.VMEM_SHARED((NPAD, width), jnp.float32),
        ],
    )
    return fn(srcI, dstI, attp, xl, xr).reshape(NCORES, NPAD, width)


def _tc_proj_kernel(x, Wl, bl, Wr, br, xl, xr):
    h = x[...]
    xl[...] = jnp.dot(h, Wl[...], preferred_element_type=jnp.float32) + bl[...]
    xr[...] = jnp.dot(h, Wr[...], preferred_element_type=jnp.float32) + br[...]


def _tc_proj(x, Wl, bl, Wr, br):
    din, w = Wl.shape
    return pl.pallas_call(
        _tc_proj_kernel,
        grid=(GRID,),
        in_specs=[
            pl.BlockSpec((RB, din), lambda i: (i, 0)),
            pl.BlockSpec((din, w), lambda i: (0, 0)),
            pl.BlockSpec((1, w), lambda i: (0, 0)),
            pl.BlockSpec((din, w), lambda i: (0, 0)),
            pl.BlockSpec((1, w), lambda i: (0, 0)),
        ],
        out_specs=[
            pl.BlockSpec((RB, w), lambda i: (i, 0)),
            pl.BlockSpec((RB, w), lambda i: (i, 0)),
        ],
        out_shape=[jax.ShapeDtypeStruct((NPAD, w), jnp.float32)] * 2,
    )(x, Wl, bl.reshape(1, -1), Wr, br.reshape(1, -1))


def _tc_mid_kernel(acc, h, rW, rb, cb, Wl, bl, Wr, br, hnew, xl, xr):
    a = acc[0] + acc[1]
    num = a[:, : H * HID]
    den = a[:, H * HID : H * HID + H]
    denb = jnp.reshape(
        jnp.broadcast_to(den.reshape(RB, 1, H), (RB, HID, H)), (RB, H * HID))
    o = num / (denb + 1e-16) + cb[...]
    hp = h[...]
    res = jnp.dot(hp, rW[...], preferred_element_type=jnp.float32) + rb[...]
    t = o + res
    hn = jnp.maximum(t, 0.2 * t)
    hnew[...] = hn
    xl[...] = jnp.dot(hn, Wl[...], preferred_element_type=jnp.float32) + bl[...]
    xr[...] = jnp.dot(hn, Wr[...], preferred_element_type=jnp.float32) + br[...]


def _tc_mid(acc, h, rW, rb, cb, Wl, bl, Wr, br):
    din = h.shape[1]
    w = Wl.shape[1]
    hd = H * HID
    return pl.pallas_call(
        _tc_mid_kernel,
        grid=(GRID,),
        in_specs=[
            pl.BlockSpec((2, RB, hd + L), lambda i: (0, i, 0)),
            pl.BlockSpec((RB, din), lambda i: (i, 0)),
            pl.BlockSpec((din, hd), lambda i: (0, 0)),
            pl.BlockSpec((1, hd), lambda i: (0, 0)),
            pl.BlockSpec((1, hd), lambda i: (0, 0)),
            pl.BlockSpec((hd, w), lambda i: (0, 0)),
            pl.BlockSpec((1, w), lambda i: (0, 0)),
            pl.BlockSpec((hd, w), lambda i: (0, 0)),
            pl.BlockSpec((1, w), lambda i: (0, 0)),
        ],
        out_specs=[
            pl.BlockSpec((RB, hd), lambda i: (i, 0)),
            pl.BlockSpec((RB, w), lambda i: (i, 0)),
            pl.BlockSpec((RB, w), lambda i: (i, 0)),
        ],
        out_shape=[
            jax.ShapeDtypeStruct((NPAD, hd), jnp.float32),
            jax.ShapeDtypeStruct((NPAD, w), jnp.float32),
            jax.ShapeDtypeStruct((NPAD, w), jnp.float32),
        ],
    )(acc, h, rW, rb.reshape(1, -1), cb.reshape(1, -1),
      Wl, bl.reshape(1, -1), Wr, br.reshape(1, -1))


def _tc_final_kernel(a0, a1, a2, a3, cb, out):
    s = None
    for acc in (a0, a1, a2, a3):
        for c in (0, 1):
            a = acc[c]
            num = a[:, :NC]
            den = a[:, W5 : W5 + 1]
            q = num / (den + 1e-16)
            s = q if s is None else s + q
    logits = s * 0.125 + cb[...]
    m = jnp.max(logits, axis=1, keepdims=True)
    z = logits - m
    out[...] = z - jnp.log(jnp.sum(jnp.exp(z), axis=1, keepdims=True))


def _tc_final(accs, cb):
    w5 = W5 + L
    return pl.pallas_call(
        _tc_final_kernel,
        grid=(GRID,),
        in_specs=[pl.BlockSpec((2, RB, w5), lambda i: (0, i, 0))] * 4 + [
            pl.BlockSpec((1, NC), lambda i: (0, 0)),
        ],
        out_specs=pl.BlockSpec((RB, NC), lambda i: (i, 0)),
        out_shape=jax.ShapeDtypeStruct((NPAD, NC), jnp.float32),
    )(*accs, cb.reshape(1, -1))


@jax.jit
def kernel(x, edge_index,
           c1_Wl, c1_bl, c1_Wr, c1_br, c1_att, c1_bias,
           c2_Wl, c2_bl, c2_Wr, c2_br, c2_att, c2_bias,
           c3_Wl, c3_bl, c3_Wr, c3_br, c3_att, c3_bias,
           c4_Wl, c4_bl, c4_Wr, c4_br, c4_att, c4_bias,
           c5_Wl, c5_bl, c5_Wr, c5_br, c5_att, c5_bias,
           r1_W, r1_b, r2_W, r2_b, r3_W, r3_b, r4_W, r4_b):
    p = dict(locals())
    perm = _PERM64
    loop = jnp.arange(N, dtype=jnp.int32)
    pad = jnp.full((EPAD - E - N,), N, jnp.int32)
    src = jnp.concatenate([edge_index[0], loop, pad])
    dst = jnp.concatenate([edge_index[1], loop, pad])
    srcI32 = src.reshape(NW, NB32, BLK)
    dstI32 = dst.reshape(NW, NB32, BLK)
    srcI16 = src.reshape(NSUB, NB16, BLK)
    dstI16 = dst.reshape(NSUB, NB16, BLK)
    xpad = jnp.pad(x, ((0, NPAD - N), (0, 0)))

    att_p = []
    for i in range(1, 5):
        att = p[f"c{i}_att"]
        att_p.append(att.T.reshape(-1))
    att5 = p["c5_att"]
    zpad8 = jnp.zeros((8,), jnp.float32)
    att5_h = [jnp.concatenate([att5[hh], zpad8]) for hh in range(H)]

    h = xpad
    for i in range(1, 5):
        Wl = p[f"c{i}_Wl"][:, perm]
        bl = p[f"c{i}_bl"][perm]
        Wr = p[f"c{i}_Wr"][:, perm]
        br = p[f"c{i}_br"][perm]
        if i > 1:
            Wl = Wl[perm]
            Wr = Wr[perm]
        if i == 1:
            xl, xr = _tc_proj(h, Wl, bl, Wr, br)
        acc = _sc_edge(xl, xr, att_p[i - 1], srcI32, dstI32, 4, 1, False)
        cb = p[f"c{i}_bias"][perm]
        rW = p[f"r{i}_W"][:, perm]
        if i > 1:
            rW = rW[perm]
        rb = p[f"r{i}_b"][perm]
        if i < 4:
            nWl = p[f"c{i+1}_Wl"][:, perm][perm]
            nbl = p[f"c{i+1}_bl"][perm]
            nWr = p[f"c{i+1}_Wr"][:, perm][perm]
            nbr = p[f"c{i+1}_br"][perm]
        else:
            Wl5 = p["c5_Wl"][perm]
            Wr5 = p["c5_Wr"][perm]
            zW = jnp.zeros((H * HID, 8), jnp.float32)
            nWl = jnp.concatenate(
                sum(([Wl5[:, hh * NC : (hh + 1) * NC], zW] for hh in range(H)),
                    []), axis=1)
            nWr = jnp.concatenate(
                sum(([Wr5[:, hh * NC : (hh + 1) * NC], zW] for hh in range(H)),
                    []), axis=1)
            nbl = jnp.concatenate(
                sum(([p["c5_bl"][hh * NC : (hh + 1) * NC], zpad8]
                     for hh in range(H)), []))
            nbr = jnp.concatenate(
                sum(([p["c5_br"][hh * NC : (hh + 1) * NC], zpad8]
                     for hh in range(H)), []))
        h, xl, xr = _tc_mid(acc, h, rW, rb, cb, nWl, nbl, nWr, nbr)

    accs = []
    for j in range(4):
        xlj = jnp.concatenate([xl[:, (2 * j) * W5 : (2 * j + 1) * W5],
                               xl[:, (2 * j + 1) * W5 : (2 * j + 2) * W5]],
                              axis=0)
        xrj = jnp.concatenate([xr[:, (2 * j) * W5 : (2 * j + 1) * W5],
                               xr[:, (2 * j + 1) * W5 : (2 * j + 2) * W5]],
                              axis=0)
        attj = jnp.concatenate([att5_h[2 * j], att5_h[2 * j + 1]])
        accs.append(_sc_edge(xlj, xrj, attj, srcI16, dstI16, 3, 4, True))
    out = _tc_final(accs, p["c5_bias"])
    return out[:N]


# speedup vs baseline: 55.3308x; 1.0852x over previous
import functools
import numpy as np
import jax
import jax.numpy as jnp
from jax import lax
from jax.experimental import pallas as pl
from jax.experimental.pallas import tpu as pltpu
from jax.experimental.pallas import tpu_sc as plsc

N = 10000
E = 320000
D = 128
H = 8
HID = 8
NC = 40

NCORES, NSUB, L = 2, 16, 16
NW = NCORES * NSUB
NPAD = 10240
BLK = 128
EPAD = 331776
NB32 = EPAD // (NW * BLK)
NB16 = EPAD // (NSUB * BLK)
RB = 1280
GRID = NPAD // RB

_PERM64 = np.array([(f % H) * HID + f // H for f in range(H * HID)], dtype=np.int32)
W5 = 48


def _fold(s, idx):
    return s + s.at[idx].get(mode="promise_in_bounds")


def _sc_edge_kernel(nvec, nfold, per_core, srcI, dstI, att_hbm,
                    xl_hbm, xr_hbm, out_hbm, src_v, dst_v, dstg0, dstg1,
                    attv, xlg0, xrg0, xlg1, xrg1, wv0, wv1,
                    s1a, s2a, s1b, s2b, s3a, s3b, acc_sh):
    cid = lax.axis_index("c")
    sid = lax.axis_index("s")
    nb = srcI.shape[-2]
    off = cid * NPAD
    if per_core:
        pltpu.sync_copy(srcI.at[sid], src_v)
        pltpu.sync_copy(dstI.at[sid], dst_v)
        pltpu.sync_copy(att_hbm.at[pl.ds(cid * nvec * L, nvec * L)], attv)
        def opre(b, _):
            for k in range(BLK // L):
                sl = pl.ds(L * k, L)
                src_v[b, sl] = src_v[b, sl] + off
            return _

        lax.fori_loop(0, nb, opre, None)
    else:
        wid = sid * NCORES + cid
        pltpu.sync_copy(srcI.at[wid], src_v)
        pltpu.sync_copy(dstI.at[wid], dst_v)
        pltpu.sync_copy(att_hbm, attv)

    nw = nvec + 1
    zero = jnp.zeros((L,), jnp.float32)
    wvs = (wv0, wv1)

    def zrow(e, _):
        for k in range(nw):
            wv0[e, pl.ds(L * k, L)] = zero
            wv1[e, pl.ds(L * k, L)] = zero
        return _

    lax.fori_loop(0, BLK, zrow, None)
    rows = NPAD // NSUB
    for k in range(rows // BLK):
        pltpu.sync_copy(wv0, acc_sh.at[pl.ds(sid * rows + k * BLK, BLK)])
    plsc.subcore_barrier()

    attk = [attv[pl.ds(L * k, L)] for k in range(nvec)]
    iota = jnp.arange(L, dtype=jnp.int32)
    rots = [(iota + r) % L for r in (8, 4, 2, 1)][:nfold]

    xlgs, xrgs = (xlg0, xlg1), (xrg0, xrg1)
    dstgs = (dstg0, dstg1)
    s1s, s2s = (s1a, s1b), (s2a, s2b)

    def issue(b, j):
        pltpu.async_copy(xl_hbm.at[src_v.at[b]], xlgs[j], s1s[j])
        if per_core:
            for k in range(BLK // L):
                sl = pl.ds(L * k, L)
                dstgs[j][sl] = dst_v[b, sl] + off
            pltpu.async_copy(xr_hbm.at[dstgs[j]], xrgs[j], s2s[j])
        else:
            pltpu.async_copy(xr_hbm.at[dst_v.at[b]], xrgs[j], s2s[j])

    def wait(j):
        pltpu.make_async_copy(xl_hbm.at[src_v.at[0]], xlgs[j], s1s[j]).wait()
        if per_core:
            pltpu.make_async_copy(
                xr_hbm.at[dstgs[j]], xrgs[j], s2s[j]).wait()
        else:
            pltpu.make_async_copy(
                xr_hbm.at[dst_v.at[0]], xrgs[j], s2s[j]).wait()

    s3s = (s3a, s3b)

    def scat_wait(j):
        pltpu.make_async_copy(
            wvs[j], acc_sh.at[dst_v.at[0]], s3s[j]).wait()

    def compute(b, j):
        xlg, xrg, wv = xlgs[j], xrgs[j], wvs[j]

        def edge(e, _):
            xs = []
            s = None
            for k in range(nvec):
                a = xlg[e, pl.ds(L * k, L)]
                t = a + xrg[e, pl.ds(L * k, L)]
                u = jnp.maximum(t, 0.2 * t)
                xs.append(a)
                p = u * attk[k]
                s = p if s is None else s + p
            for r in rots:
                s = _fold(s, r)
            ex = jnp.exp(s)
            for k in range(nvec):
                wv[e, pl.ds(L * k, L)] = xs[k] * ex
            wv[e, pl.ds(L * nvec, L)] = ex
            return _

        lax.fori_loop(0, BLK, edge, None)
        pltpu.async_copy(wv, acc_sh.at[dst_v.at[b]], s3s[j], add=True)

    last = nb - 1
    issue(0, 0)
    issue(1, 1)
    pltpu.async_copy(wv0, acc_sh.at[dst_v.at[0]], s3a, add=True)
    pltpu.async_copy(wv1, acc_sh.at[dst_v.at[0]], s3b, add=True)

    def body(i, _):
        b0 = 2 * i
        wait(0)
        scat_wait(0)
        compute(b0, 0)
        issue(jnp.minimum(b0 + 2, last), 0)
        wait(1)
        scat_wait(1)
        compute(b0 + 1, 1)
        issue(jnp.minimum(b0 + 3, last), 1)
        return _

    lax.fori_loop(0, nb // 2, body, None)
    if nb % 2:
        wait(0)
        scat_wait(0)
        compute(last, 0)
        wait(1)
    else:
        wait(0)
        wait(1)
    scat_wait(0)
    scat_wait(1)
    plsc.subcore_barrier()
    pltpu.sync_copy(acc_sh.at[pl.ds(sid * rows, rows)],
                    out_hbm.at[pl.ds(cid * NPAD + sid * rows, rows)])


def _sc_edge(xl, xr, attp, srcI, dstI, nvec, nfold, per_core):
    width = nvec * L + L
    mesh = plsc.VectorSubcoreMesh(core_axis_name="c", subcore_axis_name="s")
    nb = NB16 if per_core else NB32
    fn = pl.kernel(
        functools.partial(_sc_edge_kernel, nvec, nfold, per_core),
        out_type=jax.ShapeDtypeStruct((NCORES * NPAD, width), jnp.float32),
        mesh=mesh,
        compiler_params=pltpu.CompilerParams(use_tc_tiling_on_sc=False),
        scratch_types=[
            pltpu.VMEM((nb, BLK), jnp.int32),
            pltpu.VMEM((nb, BLK), jnp.int32),
            pltpu.VMEM((BLK,), jnp.int32),
            pltpu.VMEM((BLK,), jnp.int32),
            pltpu.VMEM((nvec * L,), jnp.float32),
            pltpu.VMEM((BLK, nvec * L), jnp.float32),
            pltpu.VMEM((BLK, nvec * L), jnp.float32),
            pltpu.VMEM((BLK, nvec * L), jnp.float32),
            pltpu.VMEM((BLK, nvec * L), jnp.float32),
            pltpu.VMEM((BLK, width), jnp.float32),
            pltpu.VMEM((BLK, width), jnp.float32),
            pltpu.SemaphoreType.DMA,
            pltpu.SemaphoreType.DMA,
            pltpu.SemaphoreType.DMA,
            pltpu.SemaphoreType.DMA,
            pltpu.SemaphoreType.DMA,
            pltpu.SemaphoreType.DMA,
            pltpu.VMEM_SHARED((NPAD, width), jnp.float32),
        ],
    )
    return fn(srcI, dstI, attp, xl, xr).reshape(NCORES, NPAD, width)


def _tc_proj_kernel(x, Wl, bl, Wr, br, xl, xr):
    h = x[...]
    xl[...] = jnp.dot(h, Wl[...], preferred_element_type=jnp.float32) + bl[...]
    xr[...] = jnp.dot(h, Wr[...], preferred_element_type=jnp.float32) + br[...]


def _tc_proj(x, Wl, bl, Wr, br):
    din, w = Wl.shape
    return pl.pallas_call(
        _tc_proj_kernel,
        grid=(GRID,),
        in_specs=[
            pl.BlockSpec((RB, din), lambda i: (i, 0)),
            pl.BlockSpec((din, w), lambda i: (0, 0)),
            pl.BlockSpec((1, w), lambda i: (0, 0)),
            pl.BlockSpec((din, w), lambda i: (0, 0)),
            pl.BlockSpec((1, w), lambda i: (0, 0)),
        ],
        out_specs=[
            pl.BlockSpec((RB, w), lambda i: (i, 0)),
            pl.BlockSpec((RB, w), lambda i: (i, 0)),
        ],
        out_shape=[jax.ShapeDtypeStruct((NPAD, w), jnp.float32)] * 2,
    )(x, Wl, bl.reshape(1, -1), Wr, br.reshape(1, -1))


def _tc_mid_kernel(acc, h, rW, rb, cb, Wl, bl, Wr, br, hnew, xl, xr):
    a = acc[0] + acc[1]
    num = a[:, : H * HID]
    den = a[:, H * HID : H * HID + H]
    denb = jnp.reshape(
        jnp.broadcast_to(den.reshape(RB, 1, H), (RB, HID, H)), (RB, H * HID))
    o = num / (denb + 1e-16) + cb[...]
    hp = h[...]
    res = jnp.dot(hp, rW[...], preferred_element_type=jnp.float32) + rb[...]
    t = o + res
    hn = jnp.maximum(t, 0.2 * t)
    hnew[...] = hn
    xl[...] = jnp.dot(hn, Wl[...], preferred_element_type=jnp.float32) + bl[...]
    xr[...] = jnp.dot(hn, Wr[...], preferred_element_type=jnp.float32) + br[...]


def _tc_mid(acc, h, rW, rb, cb, Wl, bl, Wr, br):
    din = h.shape[1]
    w = Wl.shape[1]
    hd = H * HID
    return pl.pallas_call(
        _tc_mid_kernel,
        grid=(GRID,),
        in_specs=[
            pl.BlockSpec((2, RB, hd + L), lambda i: (0, i, 0)),
            pl.BlockSpec((RB, din), lambda i: (i, 0)),
            pl.BlockSpec((din, hd), lambda i: (0, 0)),
            pl.BlockSpec((1, hd), lambda i: (0, 0)),
            pl.BlockSpec((1, hd), lambda i: (0, 0)),
            pl.BlockSpec((hd, w), lambda i: (0, 0)),
            pl.BlockSpec((1, w), lambda i: (0, 0)),
            pl.BlockSpec((hd, w), lambda i: (0, 0)),
            pl.BlockSpec((1, w), lambda i: (0, 0)),
        ],
        out_specs=[
            pl.BlockSpec((RB, hd), lambda i: (i, 0)),
            pl.BlockSpec((RB, w), lambda i: (i, 0)),
            pl.BlockSpec((RB, w), lambda i: (i, 0)),
        ],
        out_shape=[
            jax.ShapeDtypeStruct((NPAD, hd), jnp.float32),
            jax.ShapeDtypeStruct((NPAD, w), jnp.float32),
            jax.ShapeDtypeStruct((NPAD, w), jnp.float32),
        ],
    )(acc, h, rW, rb.reshape(1, -1), cb.reshape(1, -1),
      Wl, bl.reshape(1, -1), Wr, br.reshape(1, -1))


def _tc_final_kernel(a0, a1, a2, a3, cb, out):
    s = None
    for acc in (a0, a1, a2, a3):
        for c in (0, 1):
            a = acc[c]
            num = a[:, :NC]
            den = a[:, W5 : W5 + 1]
            q = num / (den + 1e-16)
            s = q if s is None else s + q
    logits = s * 0.125 + cb[...]
    m = jnp.max(logits, axis=1, keepdims=True)
    z = logits - m
    out[...] = z - jnp.log(jnp.sum(jnp.exp(z), axis=1, keepdims=True))


def _tc_final(accs, cb):
    w5 = W5 + L
    return pl.pallas_call(
        _tc_final_kernel,
        grid=(GRID,),
        in_specs=[pl.BlockSpec((2, RB, w5), lambda i: (0, i, 0))] * 4 + [
            pl.BlockSpec((1, NC), lambda i: (0, 0)),
        ],
        out_specs=pl.BlockSpec((RB, NC), lambda i: (i, 0)),
        out_shape=jax.ShapeDtypeStruct((NPAD, NC), jnp.float32),
    )(*accs, cb.reshape(1, -1))


@jax.jit
def kernel(x, edge_index,
           c1_Wl, c1_bl, c1_Wr, c1_br, c1_att, c1_bias,
           c2_Wl, c2_bl, c2_Wr, c2_br, c2_att, c2_bias,
           c3_Wl, c3_bl, c3_Wr, c3_br, c3_att, c3_bias,
           c4_Wl, c4_bl, c4_Wr, c4_br, c4_att, c4_bias,
           c5_Wl, c5_bl, c5_Wr, c5_br, c5_att, c5_bias,
           r1_W, r1_b, r2_W, r2_b, r3_W, r3_b, r4_W, r4_b):
    p = dict(locals())
    perm = _PERM64
    loop = jnp.arange(N, dtype=jnp.int32)
    pad = jnp.full((EPAD - E - N,), N, jnp.int32)
    src = jnp.concatenate([edge_index[0], loop, pad])
    dst = jnp.concatenate([edge_index[1], loop, pad])
    srcI32 = src.reshape(NW, NB32, BLK)
    dstI32 = dst.reshape(NW, NB32, BLK)
    srcI16 = src.reshape(NSUB, NB16, BLK)
    dstI16 = dst.reshape(NSUB, NB16, BLK)
    xpad = jnp.pad(x, ((0, NPAD - N), (0, 0)))

    att_p = []
    for i in range(1, 5):
        att = p[f"c{i}_att"]
        att_p.append(att.T.reshape(-1))
    att5 = p["c5_att"]
    zpad8 = jnp.zeros((8,), jnp.float32)
    att5_h = [jnp.concatenate([att5[hh], zpad8]) for hh in range(H)]

    h = xpad
    for i in range(1, 5):
        Wl = p[f"c{i}_Wl"][:, perm]
        bl = p[f"c{i}_bl"][perm]
        Wr = p[f"c{i}_Wr"][:, perm]
        br = p[f"c{i}_br"][perm]
        if i > 1:
            Wl = Wl[perm]
            Wr = Wr[perm]
        if i == 1:
            xl, xr = _tc_proj(h, Wl, bl, Wr, br)
        acc = _sc_edge(xl, xr, att_p[i - 1], srcI32, dstI32, 4, 1, False)
        cb = p[f"c{i}_bias"][perm]
        rW = p[f"r{i}_W"][:, perm]
        if i > 1:
            rW = rW[perm]
        rb = p[f"r{i}_b"][perm]
        if i < 4:
            nWl = p[f"c{i+1}_Wl"][:, perm][perm]
            nbl = p[f"c{i+1}_bl"][perm]
            nWr = p[f"c{i+1}_Wr"][:, perm][perm]
            nbr = p[f"c{i+1}_br"][perm]
        else:
            Wl5 = p["c5_Wl"][perm]
            Wr5 = p["c5_Wr"][perm]
            zW = jnp.zeros((H * HID, 8), jnp.float32)
            nWl = jnp.concatenate(
                sum(([Wl5[:, hh * NC : (hh + 1) * NC], zW] for hh in range(H)),
                    []), axis=1)
            nWr = jnp.concatenate(
                sum(([Wr5[:, hh * NC : (hh + 1) * NC], zW] for hh in range(H)),
                    []), axis=1)
            nbl = jnp.concatenate(
                sum(([p["c5_bl"][hh * NC : (hh + 1) * NC], zpad8]
                     for hh in range(H)), []))
            nbr = jnp.concatenate(
                sum(([p["c5_br"][hh * NC : (hh + 1) * NC], zpad8]
                     for hh in range(H)), []))
        h, xl, xr = _tc_mid(acc, h, rW, rb, cb, nWl, nbl, nWr, nbr)

    accs = []
    for j in range(4):
        xlj = jnp.concatenate([xl[:, (2 * j) * W5 : (2 * j + 1) * W5],
                               xl[:, (2 * j + 1) * W5 : (2 * j + 2) * W5]],
                              axis=0)
        xrj = jnp.concatenate([xr[:, (2 * j) * W5 : (2 * j + 1) * W5],
                               xr[:, (2 * j + 1) * W5 : (2 * j + 2) * W5]],
                              axis=0)
        attj = jnp.concatenate([att5_h[2 * j], att5_h[2 * j + 1]])
        accs.append(_sc_edge(xlj, xrj, attj, srcI16, dstI16, 3, 4, True))
    out = _tc_final(accs, p["c5_bias"])
    return out[:N]
